# baseline (speedup 1.0000x reference)
import functools

import jax
import jax.numpy as jnp
from jax.experimental import pallas as pl
from jax.experimental.pallas import tpu as pltpu

TOPK = 8
NUM_EXPERTS = 64


def _router_body(x_ref, w_ref, idx_ref, val_ref):
    logits = jnp.dot(x_ref[...], w_ref[...], preferred_element_type=jnp.float32)
    iota = jax.lax.broadcasted_iota(jnp.int32, logits.shape, 1)
    bits = jax.lax.bitcast_convert_type(logits, jnp.int32)
    idxfield = jnp.where(bits >= 0, 63 - iota, iota)
    keys = jax.lax.bitcast_convert_type((bits & ~63) | idxfield, jnp.float32)
    kmaxes = []
    neg_inf = jnp.float32(-jnp.inf)
    for _ in range(TOPK):
        kmax = jnp.max(keys, axis=-1, keepdims=True)
        kmaxes.append(kmax)
        keys = jnp.where(keys == kmax, neg_inf, keys)
    kcat = jnp.concatenate(kmaxes, axis=-1)
    kbits = jax.lax.bitcast_convert_type(kcat, jnp.int32)
    low = kbits & 63
    idx_ref[...] = jnp.where(kbits >= 0, 63 - low, low)
    lsel = jax.lax.bitcast_convert_type(kbits & ~63, jnp.float32)
    m = lsel[:, 0:1]
    s = jnp.sum(jnp.exp(logits - m), axis=-1, keepdims=True)
    val_ref[...] = jnp.exp(lsel - m) / s


@functools.partial(jax.jit, static_argnames=("interpret",))
def kernel(x, W_g, interpret=False):
    n_tokens, d_hidden = x.shape
    n_experts = W_g.shape[1]
    bt = 256
    grid = (n_tokens // bt,)
    idx, val = pl.pallas_call(
        _router_body,
        grid=grid,
        in_specs=[
            pl.BlockSpec((bt, d_hidden), lambda i: (i, 0)),
            pl.BlockSpec((d_hidden, n_experts), lambda i: (0, 0)),
        ],
        out_specs=[
            pl.BlockSpec((bt, TOPK), lambda i: (i, 0)),
            pl.BlockSpec((bt, TOPK), lambda i: (i, 0)),
        ],
        out_shape=[
            jax.ShapeDtypeStruct((n_tokens, TOPK), jnp.int32),
            jax.ShapeDtypeStruct((n_tokens, TOPK), jnp.float32),
        ],
        compiler_params=pltpu.CompilerParams(
            dimension_semantics=("parallel",),
        ),
        interpret=interpret,
    )(x, W_g)
    return (idx, val)


# speedup vs baseline: 1.3545x; 1.3545x over previous
import functools

import jax
import jax.numpy as jnp
from jax.experimental import pallas as pl
from jax.experimental.pallas import tpu as pltpu

TOPK = 8
NUM_EXPERTS = 64


def _router_body(x1_ref, x2_ref, w_ref, idx_ref, val_ref):
    kh = x1_ref.shape[1]
    logits = (
        jnp.dot(x1_ref[...], w_ref[0:kh, :], preferred_element_type=jnp.float32)
        + jnp.dot(x2_ref[...], w_ref[kh:, :], preferred_element_type=jnp.float32)
    )
    iota = jax.lax.broadcasted_iota(jnp.int32, logits.shape, 1)
    bits = jax.lax.bitcast_convert_type(logits, jnp.int32)
    idxfield = jnp.where(bits >= 0, 63 - iota, iota)
    keys = jax.lax.bitcast_convert_type((bits & ~63) | idxfield, jnp.float32)
    kmaxes = []
    neg_inf = jnp.float32(-jnp.inf)
    for _ in range(TOPK):
        kmax = jnp.max(keys, axis=-1, keepdims=True)
        kmaxes.append(kmax)
        keys = jnp.where(keys == kmax, neg_inf, keys)
    kcat = jnp.concatenate(kmaxes, axis=-1)
    kbits = jax.lax.bitcast_convert_type(kcat, jnp.int32)
    low = kbits & 63
    idx_ref[...] = jnp.where(kbits >= 0, 63 - low, low)
    lsel = jax.lax.bitcast_convert_type(kbits & ~63, jnp.float32)
    m = lsel[:, 0:1]
    s = jnp.sum(jnp.exp(logits - m), axis=-1, keepdims=True)
    val_ref[...] = jnp.exp(lsel - m) / s


@functools.partial(jax.jit, static_argnames=("interpret",))
def kernel(x, W_g, interpret=False):
    n_tokens, d_hidden = x.shape
    n_experts = W_g.shape[1]
    bt = 1024
    kh = d_hidden // 2
    grid = (n_tokens // bt,)
    idx, val = pl.pallas_call(
        _router_body,
        grid=grid,
        in_specs=[
            pl.BlockSpec((bt, kh), lambda i: (i, 0)),
            pl.BlockSpec((bt, kh), lambda i: (i, 1)),
            pl.BlockSpec((d_hidden, n_experts), lambda i: (0, 0)),
        ],
        out_specs=[
            pl.BlockSpec((bt, TOPK), lambda i: (i, 0)),
            pl.BlockSpec((bt, TOPK), lambda i: (i, 0)),
        ],
        out_shape=[
            jax.ShapeDtypeStruct((n_tokens, TOPK), jnp.int32),
            jax.ShapeDtypeStruct((n_tokens, TOPK), jnp.float32),
        ],
        compiler_params=pltpu.CompilerParams(
            dimension_semantics=("arbitrary",),
        ),
        interpret=interpret,
    )(x, x, W_g)
    return (idx, val)
